# baseline (speedup 1.0000x reference)
import functools

import jax
import jax.numpy as jnp
from jax import lax
from jax.experimental import pallas as pl
from jax.experimental.pallas import tpu as pltpu
from jax.experimental.pallas import tpu_sc as plsc

VOCAB = 1000000
EMBED = 64
CTX = 200
BATCH = 4096
SEQ = 200

N_WORKERS = 32
BW = BATCH // N_WORKERS
N_IT = SEQ
NBB = BW // 16


def _worker_id():
    return lax.axis_index("s") * 2 + lax.axis_index("c")


def _body(idx_hbm, table_hbm, pos_hbm, out_hbm, pos_v,
          idx0, gidx0, poff0, buf0, obuf0,
          idx1, gidx1, poff1, buf1, obuf1,
          gsem0, gsem1, ssem0, ssem1, isem0, isem1):
    wid = _worker_id()
    bw0 = wid * BW

    pltpu.sync_copy(pos_hbm, pos_v)

    def fire_idx(i, idx, isem):
        s = jnp.minimum(i, N_IT - 1)
        pltpu.async_copy(idx_hbm.at[s, pl.ds(bw0, BW)], idx, isem)

    def wait_idx(idx, isem):
        pltpu.make_async_copy(idx_hbm.at[0, pl.ds(bw0, BW)], idx, isem).wait()

    def prep_gather(idx, gidx, poff):
        for c in range(NBB):
            sl = pl.ds(c * 16, 16)
            v = idx[sl]
            gidx[sl] = lax.shift_right_logical(v, 1)
            poff[sl] = lax.shift_left(v & 1, 6)

    def fire_gather(gidx, buf, gsem):
        pltpu.async_copy(table_hbm.at[gidx], buf, gsem)

    def drain_gather(gidx, buf, gsem):
        pltpu.make_async_copy(table_hbm.at[gidx], buf, gsem).wait()

    def compute(i, poff, buf, obuf):
        rows = [lax.iota(jnp.int32, 16) + (bb * 16) for bb in range(NBB)]
        pcs = [poff[pl.ds(bb * 16, 16)] for bb in range(NBB)]
        pbase = i * EMBED

        @plsc.parallel_loop(0, EMBED, 1, unroll=2)
        def _d(d):
            bc = plsc.load_gather(pos_v, [jnp.full((16,), pbase + d, jnp.int32)])
            for bb in range(NBB):
                val = plsc.load_gather(buf, [rows[bb], pcs[bb] + d])
                obuf[d, pl.ds(bb * 16, 16)] = val + bc

    def fire_store(i, obuf, ssem):
        pltpu.async_copy(obuf, out_hbm.at[i, :, pl.ds(bw0, BW)], ssem)

    def wait_store(obuf, ssem):
        pltpu.make_async_copy(obuf, out_hbm.at[0, :, pl.ds(bw0, BW)], ssem).wait()

    s0 = (idx0, gidx0, poff0, buf0, obuf0, gsem0, ssem0, isem0)
    s1 = (idx1, gidx1, poff1, buf1, obuf1, gsem1, ssem1, isem1)

    def steady(i, X, Y):
        (idxX, gidxX, poffX, bufX, obufX, gsemX, ssemX, isemX) = X
        (idxY, gidxY, poffY, bufY, obufY, gsemY, ssemY, isemY) = Y
        wait_store(obufY, ssemY)
        wait_idx(idxY, isemY)
        prep_gather(idxY, gidxY, poffY)
        fire_gather(gidxY, bufY, gsemY)
        drain_gather(gidxX, bufX, gsemX)
        fire_idx(i + 2, idxX, isemX)
        compute(i, poffX, bufX, obufX)
        fire_store(i, obufX, ssemX)

    fire_idx(0, idx0, isem0)
    wait_idx(idx0, isem0)
    prep_gather(idx0, gidx0, poff0)
    fire_gather(gidx0, buf0, gsem0)
    fire_idx(1, idx1, isem1)

    wait_idx(idx1, isem1)
    prep_gather(idx1, gidx1, poff1)
    fire_gather(gidx1, buf1, gsem1)
    drain_gather(gidx0, buf0, gsem0)
    fire_idx(2, idx0, isem0)
    compute(0, poff0, buf0, obuf0)
    fire_store(0, obuf0, ssem0)

    def pair(t, _):
        i = 2 * t + 1
        steady(i, s1, s0)
        steady(i + 1, s0, s1)
        return 0

    lax.fori_loop(0, (N_IT - 2) // 2, pair, 0)

    wait_store(obuf0, ssem0)
    drain_gather(gidx1, buf1, gsem1)
    compute(N_IT - 1, poff1, buf1, obuf1)
    fire_store(N_IT - 1, obuf1, ssem1)
    wait_idx(idx0, isem0)
    wait_store(obuf1, ssem1)


@jax.jit
def kernel(token_ids, token_emb, pos_emb):
    ids_t = token_ids.T.astype(jnp.int32)
    tbl = token_emb.reshape(VOCAB // 2, 2 * EMBED)
    posf = pos_emb.reshape(CTX * EMBED)
    mesh = plsc.VectorSubcoreMesh(core_axis_name="c", subcore_axis_name="s")

    def slot_scratch():
        return [
            pltpu.VMEM((BW,), jnp.int32),
            pltpu.VMEM((BW,), jnp.int32),
            pltpu.VMEM((BW,), jnp.int32),
            pltpu.VMEM((BW, 2 * EMBED), jnp.float32),
            pltpu.VMEM((EMBED, BW), jnp.float32),
        ]

    out = pl.kernel(
        _body,
        out_type=jax.ShapeDtypeStruct((SEQ, EMBED, BATCH), jnp.float32),
        mesh=mesh,
        compiler_params=pltpu.CompilerParams(needs_layout_passes=False),
        scratch_types=[
            pltpu.VMEM((CTX * EMBED,), jnp.float32),
            *slot_scratch(),
            *slot_scratch(),
            pltpu.SemaphoreType.DMA,
            pltpu.SemaphoreType.DMA,
            pltpu.SemaphoreType.DMA,
            pltpu.SemaphoreType.DMA,
            pltpu.SemaphoreType.DMA,
            pltpu.SemaphoreType.DMA,
        ],
    )(ids_t, tbl, posf)
    return out.transpose(2, 0, 1)
